# baseline (speedup 1.0000x reference)
import jax
import jax.numpy as jnp
from jax import lax
from jax.experimental import pallas as pl
from jax.experimental.pallas import tpu as pltpu
from jax.experimental.pallas import tpu_sc as plsc

N = 320000
D = 128
S = 1024
L = 16
NC = 2
NS = 16
NW = NC * NS
SPW = S // NW
C = 256
GPC = C // L
NK = D // L

_PR = 4.5
_PC = (1.6489050394e+00, 4.1180726077e-01, 5.0446300670e-02,
       -2.9469614882e-02, -7.2438226860e-03, 2.1147717633e-03,
       6.9129190923e-04, -1.0996859341e-04, -4.0963961651e-05,
       3.3960180491e-06, 1.3316164224e-06, -4.5173916069e-08,
       -1.7941291354e-08)


def _sc_body(x_hbm, b_hbm, bnd_hbm, wb_hbm, out_hbm,
             xbuf, bbuf, xbuf1, bbuf1, bndv, wbv,
             acc_sum, acc_max, acc_se, outbuf,
             semx0, semb0, semx1, semb1):
    cid = lax.axis_index("c")
    sid = lax.axis_index("s")
    wid = (cid * NS + sid).astype(jnp.int32)
    seg_base = wid * SPW

    pltpu.sync_copy(bnd_hbm, bndv)
    pltpu.sync_copy(wb_hbm, wbv)

    wreg = [wbv[pl.ds(16 * k, L)] for k in range(NK)]
    bias = wbv[pl.ds(D, L)]

    def get_bound(j):
        return bndv[pl.ds(j, L)][0].astype(jnp.int32)

    lo = get_bound(wid)
    hi = get_bound(wid + 1)
    a0 = (lo // C) * C
    nchunks = (hi - a0 + (C - 1)) // C

    zero = jnp.zeros((L,), jnp.float32)
    ninf = jnp.full((L,), -jnp.inf, jnp.float32)

    for s in range(SPW + 1):
        for k in range(NK):
            acc_sum[s, pl.ds(16 * k, L)] = zero
            acc_max[s, pl.ds(16 * k, L)] = ninf
        acc_se[s, :] = zero

    lanes = lax.iota(jnp.int32, L)
    perm = [lanes ^ s for s in (1, 2, 4, 8)]
    _dnums = lax.GatherDimensionNumbers(
        offset_dims=(), collapsed_slice_dims=(0,), start_index_map=(0,))

    def shuffle(v, pm):
        return lax.gather(v, pm[:, None], _dnums, slice_sizes=(1,),
                          mode=lax.GatherScatterMode.PROMISE_IN_BOUNDS)

    def row_vals(xb, row):
        xv = [xb[row, pl.ds(16 * k, L)] for k in range(NK)]
        p01 = xv[0] * wreg[0] + xv[1] * wreg[1]
        p23 = xv[2] * wreg[2] + xv[3] * wreg[3]
        p45 = xv[4] * wreg[4] + xv[5] * wreg[5]
        p67 = xv[6] * wreg[6] + xv[7] * wreg[7]
        t = (p01 + p23) + (p45 + p67)
        for pm in perm:
            t = t + shuffle(t, pm)
        t = jnp.clip(t + bias, -_PR, _PR)
        e = jnp.full((L,), jnp.float32(_PC[-1]))
        for c in _PC[-2::-1]:
            e = e * t + jnp.float32(c)
        return xv, e

    def rmw(slot, se, ss, mm):
        for k in range(NK):
            acc_sum[slot, pl.ds(16 * k, L)] = (
                acc_sum[slot, pl.ds(16 * k, L)] + ss[k])
            acc_max[slot, pl.ds(16 * k, L)] = jnp.maximum(
                acc_max[slot, pl.ds(16 * k, L)], mm[k])
        acc_se[slot, :] = acc_se[slot, :] + se

    def to_slot(bval):
        lsl = bval - seg_base
        ok = (lsl >= 0) & (lsl < SPW)
        return jnp.where(ok, lsl, jnp.int32(SPW))

    def process_chunk(xb, bb):
        def group_body(g, _):
            bvec = bb[pl.ds(g * L, L)]
            b_first = bvec[0]
            b_last = bvec[L - 1]
            row0 = g * L

            def uniform_case():
                xv, e = row_vals(xb, row0)
                se = e
                ss = [e * xv[k] for k in range(NK)]
                mm = xv
                for j in range(1, L):
                    xv, e = row_vals(xb, row0 + j)
                    se = se + e
                    ss = [ss[k] + e * xv[k] for k in range(NK)]
                    mm = [jnp.maximum(mm[k], xv[k]) for k in range(NK)]
                rmw(to_slot(b_first), se, ss, mm)

            def mixed_case():
                for j in range(L):
                    xv, e = row_vals(xb, row0 + j)
                    rmw(to_slot(bvec[j]), e,
                        [e * xv[k] for k in range(NK)], xv)

            lax.cond(b_first == b_last, uniform_case, mixed_case)
            return 0

        lax.fori_loop(0, GPC, group_body, 0)

    bufs = ((xbuf, bbuf, semx0, semb0), (xbuf1, bbuf1, semx1, semb1))

    def copies(ci, xb, bb, sx, sb):
        r0 = pl.multiple_of(a0 + ci * C, C)
        return (pltpu.make_async_copy(x_hbm.at[pl.ds(r0, C)], xb, sx),
                pltpu.make_async_copy(b_hbm.at[pl.ds(r0, C)], bb, sb))

    def start(ci, xb, bb, sx, sb):
        for cp in copies(ci, xb, bb, sx, sb):
            cp.start()

    def wait(ci, xb, bb, sx, sb):
        for cp in copies(ci, xb, bb, sx, sb):
            cp.wait()

    for p in range(2):
        @pl.when(p < nchunks)
        def _(p=p):
            start(jnp.int32(p), *bufs[p])

    def pair_body(pi, _):
        ci0 = pi * 2
        for p in range(2):
            ci = ci0 + p
            xb, bb, sx, sb = bufs[p]

            @pl.when(ci < nchunks)
            def _(ci=ci, xb=xb, bb=bb, sx=sx, sb=sb):
                wait(ci, xb, bb, sx, sb)
                process_chunk(xb, bb)

                @pl.when(ci + 2 < nchunks)
                def _():
                    start(ci + 2, xb, bb, sx, sb)
        return 0

    lax.fori_loop(0, (nchunks + 1) // 2, pair_body, 0)

    for s in range(SPW):
        sev = acc_se[s, :] + 1e-16
        for k in range(NK):
            outbuf[s, pl.ds(16 * k, L)] = acc_max[s, pl.ds(16 * k, L)]
            outbuf[s, pl.ds(D + 16 * k, L)] = (
                acc_sum[s, pl.ds(16 * k, L)] / sev)
    pltpu.sync_copy(outbuf, out_hbm.at[pl.ds(pl.multiple_of(seg_base, SPW),
                                             SPW)])


@jax.jit
def _run(x, batch32, bounds, wb):
    mesh = plsc.VectorSubcoreMesh(core_axis_name="c", subcore_axis_name="s")
    fn = pl.kernel(
        _sc_body,
        out_type=jax.ShapeDtypeStruct((S, 2 * D), jnp.float32),
        mesh=mesh,
        scratch_types=[
            pltpu.VMEM((C, D), jnp.float32),
            pltpu.VMEM((C,), jnp.int32),
            pltpu.VMEM((C, D), jnp.float32),
            pltpu.VMEM((C,), jnp.int32),
            pltpu.VMEM((4 * L,), jnp.float32),
            pltpu.VMEM((D + L,), jnp.float32),
            pltpu.VMEM((SPW + 1, D), jnp.float32),
            pltpu.VMEM((SPW + 1, D), jnp.float32),
            pltpu.VMEM((SPW + 1, L), jnp.float32),
            pltpu.VMEM((SPW, 2 * D), jnp.float32),
            pltpu.SemaphoreType.DMA,
            pltpu.SemaphoreType.DMA,
            pltpu.SemaphoreType.DMA,
            pltpu.SemaphoreType.DMA,
        ],
    )
    return fn(x, batch32, bounds, wb)


def kernel(x, batch, W, b):
    batch32 = batch.astype(jnp.int32)
    targets = jnp.arange(0, S + 1, SPW, dtype=jnp.int32)
    bounds = jnp.searchsorted(batch32, targets).astype(jnp.float32)
    bounds = jnp.concatenate(
        [bounds, jnp.zeros((4 * L - (NW + 1),), jnp.float32)])
    wb = jnp.concatenate([W.reshape(D), jnp.full((L,), b[0], jnp.float32)])
    return _run(x, batch32, bounds, wb)


# speedup vs baseline: 2.0980x; 2.0980x over previous
import jax
import jax.numpy as jnp
from jax import lax
from jax.experimental import pallas as pl
from jax.experimental.pallas import tpu as pltpu
from jax.experimental.pallas import tpu_sc as plsc

N = 320000
D = 128
S = 1024
L = 16
NC = 2
NS = 16
NW = NC * NS
SPW = S // NW
C = 256
GPC = C // L
NK = D // L


def _sc_body(x_hbm, b_hbm, bnd_hbm, wb_hbm, out_hbm,
             xbuf, bbuf, xbuf1, bbuf1, bndv, wbv,
             acc_sum, acc_max, acc_se, outbuf,
             semx0, semb0, semx1, semb1):
    cid = lax.axis_index("c")
    sid = lax.axis_index("s")
    wid = (cid * NS + sid).astype(jnp.int32)
    seg_base = wid * SPW

    pltpu.sync_copy(bnd_hbm, bndv)
    pltpu.sync_copy(wb_hbm, wbv)

    wreg = [wbv[pl.ds(16 * k, L)] for k in range(NK)]
    bias = wbv[pl.ds(D, L)]

    def get_bound(j):
        return bndv[pl.ds(j, L)][0].astype(jnp.int32)

    lo = get_bound(wid)
    hi = get_bound(wid + 1)
    a0 = (lo // (2 * C)) * (2 * C)
    npairs = (hi - a0 + (2 * C - 1)) // (2 * C)

    zero = jnp.zeros((L,), jnp.float32)
    ninf = jnp.full((L,), -jnp.inf, jnp.float32)

    for s in range(SPW + 1):
        for k in range(NK):
            acc_sum[s, pl.ds(16 * k, L)] = zero
            acc_max[s, pl.ds(16 * k, L)] = ninf
        acc_se[s, :] = zero

    lanes = lax.iota(jnp.int32, L)

    def load_row(xb, row):
        return [xb[row, pl.ds(16 * k, L)] for k in range(NK)]

    def dot_partial(xv):
        p01 = xv[0] * wreg[0] + xv[1] * wreg[1]
        p23 = xv[2] * wreg[2] + xv[3] * wreg[3]
        p45 = xv[4] * wreg[4] + xv[5] * wreg[5]
        p67 = xv[6] * wreg[6] + xv[7] * wreg[7]
        return (p01 + p23) + (p45 + p67)

    _dnums = lax.GatherDimensionNumbers(
        offset_dims=(), collapsed_slice_dims=(0,), start_index_map=(0,))
    shuf_idx = {d: lanes ^ d for d in (1, 2, 4, 8)}
    half_mask = {d: (lanes & d) == 0 for d in (1, 2, 4, 8)}

    def shuffle(v, d):
        return lax.gather(v, shuf_idx[d][:, None], _dnums, slice_sizes=(1,),
                          mode=lax.GatherScatterMode.PROMISE_IN_BOUNDS)

    def combine(u, v, d):
        m = half_mask[d]
        return jnp.where(m, u, v) + shuffle(jnp.where(m, v, u), d)

    def group_evec(xb, row0):
        vs = [dot_partial(load_row(xb, row0 + j)) for j in range(L)]
        for d in (1, 2, 4, 8):
            vs = [combine(vs[2 * i], vs[2 * i + 1], d)
                  for i in range(len(vs) // 2)]
        t = vs[0] + bias
        return jnp.exp(1.0 / (1.0 + jnp.exp(-t)))

    def rmw(slot, se, ss, mm):
        for k in range(NK):
            acc_sum[slot, pl.ds(16 * k, L)] = (
                acc_sum[slot, pl.ds(16 * k, L)] + ss[k])
            acc_max[slot, pl.ds(16 * k, L)] = jnp.maximum(
                acc_max[slot, pl.ds(16 * k, L)], mm[k])
        acc_se[slot, :] = acc_se[slot, :] + se

    def to_slot(bval):
        lsl = bval - seg_base
        ok = (lsl >= 0) & (lsl < SPW)
        return jnp.where(ok, lsl, jnp.int32(SPW))

    def process_chunk(xb, bb):
        def group_body(g, _):
            bvec = bb[pl.ds(g * L, L)]
            b_first = bvec[0]
            b_last = bvec[L - 1]
            row0 = g * L

            sevec = group_evec(xb, row0)

            def eget(j):
                return jnp.full((L,), sevec[j])

            def uniform_case():
                xv = load_row(xb, row0)
                ej = eget(0)
                ss = [ej * xv[k] for k in range(NK)]
                mm = xv
                for j in range(1, L):
                    xv = load_row(xb, row0 + j)
                    ej = eget(j)
                    ss = [ss[k] + ej * xv[k] for k in range(NK)]
                    mm = [jnp.maximum(mm[k], xv[k]) for k in range(NK)]
                rmw(to_slot(b_first), sevec, ss, mm)

            def mixed_case():
                zero16 = jnp.zeros((L,), jnp.float32)
                for j in range(L):
                    xv = load_row(xb, row0 + j)
                    ej = eget(j)
                    sej = jnp.where(lanes == j, sevec, zero16)
                    rmw(to_slot(bvec[j]), sej,
                        [ej * xv[k] for k in range(NK)], xv)

            lax.cond(b_first == b_last, uniform_case, mixed_case)
            return 0

        lax.fori_loop(0, GPC, group_body, 0)

    bufs = ((xbuf, bbuf, semx0, semb0), (xbuf1, bbuf1, semx1, semb1))

    def copies(ci, xb, bb, sx, sb):
        r0 = pl.multiple_of(a0 + ci * C, C)
        return (pltpu.make_async_copy(x_hbm.at[pl.ds(r0, C)], xb, sx),
                pltpu.make_async_copy(b_hbm.at[pl.ds(r0, C)], bb, sb))

    def start(ci, xb, bb, sx, sb):
        for cp in copies(ci, xb, bb, sx, sb):
            cp.start()

    def wait(ci, xb, bb, sx, sb):
        for cp in copies(ci, xb, bb, sx, sb):
            cp.wait()

    @pl.when(npairs > 0)
    def _():
        start(jnp.int32(0), *bufs[0])
        start(jnp.int32(1), *bufs[1])

    def pair_body(pi, _):
        for p in range(2):
            ci = pi * 2 + p
            xb, bb, sx, sb = bufs[p]
            wait(ci, xb, bb, sx, sb)
            process_chunk(xb, bb)

            @pl.when(pi + 1 < npairs)
            def _(ci=ci, xb=xb, bb=bb, sx=sx, sb=sb):
                start(ci + 2, xb, bb, sx, sb)
        return 0

    lax.fori_loop(0, npairs, pair_body, 0)

    for s in range(SPW):
        sev = acc_se[s, :] + 1e-16
        for k in range(NK):
            outbuf[s, pl.ds(16 * k, L)] = acc_max[s, pl.ds(16 * k, L)]
            outbuf[s, pl.ds(D + 16 * k, L)] = (
                acc_sum[s, pl.ds(16 * k, L)] / sev)
    pltpu.sync_copy(outbuf, out_hbm.at[pl.ds(pl.multiple_of(seg_base, SPW),
                                             SPW)])


@jax.jit
def _run(x, batch32, bounds, wb):
    mesh = plsc.VectorSubcoreMesh(core_axis_name="c", subcore_axis_name="s")
    fn = pl.kernel(
        _sc_body,
        out_type=jax.ShapeDtypeStruct((S, 2 * D), jnp.float32),
        mesh=mesh,
        scratch_types=[
            pltpu.VMEM((C, D), jnp.float32),
            pltpu.VMEM((C,), jnp.int32),
            pltpu.VMEM((C, D), jnp.float32),
            pltpu.VMEM((C,), jnp.int32),
            pltpu.VMEM((4 * L,), jnp.float32),
            pltpu.VMEM((D + L,), jnp.float32),
            pltpu.VMEM((SPW + 1, D), jnp.float32),
            pltpu.VMEM((SPW + 1, D), jnp.float32),
            pltpu.VMEM((SPW + 1, L), jnp.float32),
            pltpu.VMEM((SPW, 2 * D), jnp.float32),
            pltpu.SemaphoreType.DMA,
            pltpu.SemaphoreType.DMA,
            pltpu.SemaphoreType.DMA,
            pltpu.SemaphoreType.DMA,
        ],
    )
    return fn(x, batch32, bounds, wb)


def kernel(x, batch, W, b):
    batch32 = batch.astype(jnp.int32)
    targets = jnp.arange(0, S + 1, SPW, dtype=jnp.int32)
    bounds = jnp.searchsorted(batch32, targets).astype(jnp.float32)
    bounds = jnp.concatenate(
        [bounds, jnp.zeros((4 * L - (NW + 1),), jnp.float32)])
    wb = jnp.concatenate([W.reshape(D), jnp.full((L,), b[0], jnp.float32)])
    return _run(x, batch32, bounds, wb)


# speedup vs baseline: 2.1035x; 1.0026x over previous
import jax
import jax.numpy as jnp
from jax import lax
from jax.experimental import pallas as pl
from jax.experimental.pallas import tpu as pltpu
from jax.experimental.pallas import tpu_sc as plsc

N = 320000
D = 128
S = 1024
L = 16
NC = 2
NS = 16
NW = NC * NS
SPW = S // NW
C = 256
GPC = C // L
NK = D // L


def _sc_body(x_hbm, b_hbm, bnd_hbm, wb_hbm, out_hbm,
             xbuf, bbuf, xbuf1, bbuf1, bndv, wbv,
             acc_sum, acc_max, acc_se, outbuf,
             semx0, semb0, semx1, semb1):
    cid = lax.axis_index("c")
    sid = lax.axis_index("s")
    wid = (cid * NS + sid).astype(jnp.int32)
    seg_base = wid * SPW

    pltpu.sync_copy(bnd_hbm, bndv)
    pltpu.sync_copy(wb_hbm, wbv)

    wreg = [wbv[pl.ds(16 * k, L)] for k in range(NK)]
    bias = wbv[pl.ds(D, L)]

    def get_bound(j):
        return bndv[pl.ds(j, L)][0].astype(jnp.int32)

    lo = get_bound(wid)
    hi = get_bound(wid + 1)
    a0 = (lo // (2 * C)) * (2 * C)
    npairs = (hi - a0 + (2 * C - 1)) // (2 * C)

    zero = jnp.zeros((L,), jnp.float32)
    ninf = jnp.full((L,), -jnp.inf, jnp.float32)

    for s in range(SPW + 1):
        for k in range(NK):
            acc_sum[s, pl.ds(16 * k, L)] = zero
            acc_max[s, pl.ds(16 * k, L)] = ninf
        acc_se[s, :] = zero

    lanes = lax.iota(jnp.int32, L)

    def load_row(xb, row):
        return [xb[row, pl.ds(16 * k, L)] for k in range(NK)]

    def dot_partial(xv):
        p01 = xv[0] * wreg[0] + xv[1] * wreg[1]
        p23 = xv[2] * wreg[2] + xv[3] * wreg[3]
        p45 = xv[4] * wreg[4] + xv[5] * wreg[5]
        p67 = xv[6] * wreg[6] + xv[7] * wreg[7]
        return (p01 + p23) + (p45 + p67)

    _dnums = lax.GatherDimensionNumbers(
        offset_dims=(), collapsed_slice_dims=(0,), start_index_map=(0,))
    shuf_idx = {d: lanes ^ d for d in (1, 2, 4, 8)}
    half_mask = {d: (lanes & d) == 0 for d in (1, 2, 4, 8)}

    def shuffle(v, idx):
        return lax.gather(v, idx[:, None], _dnums, slice_sizes=(1,),
                          mode=lax.GatherScatterMode.PROMISE_IN_BOUNDS)

    def combine(u, v, d):
        m = half_mask[d]
        return jnp.where(m, u, v) + shuffle(jnp.where(m, v, u), shuf_idx[d])

    def group_evec(xb, row0):
        vs = [dot_partial(load_row(xb, row0 + j)) for j in range(L)]
        for d in (1, 2, 4, 8):
            vs = [combine(vs[2 * i], vs[2 * i + 1], d)
                  for i in range(len(vs) // 2)]
        t = vs[0] + bias
        return jnp.exp(1.0 / (1.0 + jnp.exp(-t)))

    def rmw(slot, se, ss, mm):
        for k in range(NK):
            acc_sum[slot, pl.ds(16 * k, L)] = (
                acc_sum[slot, pl.ds(16 * k, L)] + ss[k])
            acc_max[slot, pl.ds(16 * k, L)] = jnp.maximum(
                acc_max[slot, pl.ds(16 * k, L)], mm[k])
        acc_se[slot, :] = acc_se[slot, :] + se

    def to_slot(bval):
        lsl = bval - seg_base
        ok = (lsl >= 0) & (lsl < SPW)
        return jnp.where(ok, lsl, jnp.int32(SPW))

    def process_chunk(xb, bb):
        def group_body(g, _):
            bvec = bb[pl.ds(g * L, L)]
            b_first = bvec[0]
            b_last = bvec[L - 1]
            row0 = g * L

            sevec = group_evec(xb, row0)

            def eget(j):
                return jnp.full((L,), sevec[j])

            def uniform_case():
                xv = load_row(xb, row0)
                ej = eget(0)
                ss = [ej * xv[k] for k in range(NK)]
                mm = xv
                for j in range(1, L):
                    xv = load_row(xb, row0 + j)
                    ej = eget(j)
                    ss = [ss[k] + ej * xv[k] for k in range(NK)]
                    mm = [jnp.maximum(mm[k], xv[k]) for k in range(NK)]
                rmw(to_slot(b_first), sevec, ss, mm)

            def mixed_case():
                for j in range(L):
                    xv = load_row(xb, row0 + j)
                    ej = eget(j)
                    sej = jnp.where(lanes == j, sevec, zero)
                    rmw(to_slot(bvec[j]), sej,
                        [ej * xv[k] for k in range(NK)], xv)

            lax.cond(b_first == b_last, uniform_case, mixed_case)
            return 0

        lax.fori_loop(0, GPC, group_body, 0)

    bufs = ((xbuf, bbuf, semx0, semb0), (xbuf1, bbuf1, semx1, semb1))

    def copies(ci, xb, bb, sx, sb):
        r0 = pl.multiple_of(a0 + ci * C, C)
        return (pltpu.make_async_copy(x_hbm.at[pl.ds(r0, C)], xb, sx),
                pltpu.make_async_copy(b_hbm.at[pl.ds(r0, C)], bb, sb))

    def start(ci, xb, bb, sx, sb):
        for cp in copies(ci, xb, bb, sx, sb):
            cp.start()

    def wait(ci, xb, bb, sx, sb):
        for cp in copies(ci, xb, bb, sx, sb):
            cp.wait()

    @pl.when(npairs > 0)
    def _():
        start(jnp.int32(0), *bufs[0])
        start(jnp.int32(1), *bufs[1])

    def pair_body(pi, _):
        for p in range(2):
            ci = pi * 2 + p
            xb, bb, sx, sb = bufs[p]
            wait(ci, xb, bb, sx, sb)
            process_chunk(xb, bb)

            @pl.when(pi + 1 < npairs)
            def _(ci=ci, xb=xb, bb=bb, sx=sx, sb=sb):
                start(ci + 2, xb, bb, sx, sb)
        return 0

    lax.fori_loop(0, npairs, pair_body, 0)

    for s in range(SPW):
        sev = acc_se[s, :]
        for d in (1, 2, 4, 8):
            sev = sev + shuffle(sev, shuf_idx[d])
        sev = sev + 1e-16
        for k in range(NK):
            outbuf[s, pl.ds(16 * k, L)] = acc_max[s, pl.ds(16 * k, L)]
            outbuf[s, pl.ds(D + 16 * k, L)] = (
                acc_sum[s, pl.ds(16 * k, L)] / sev)
    pltpu.sync_copy(outbuf, out_hbm.at[pl.ds(pl.multiple_of(seg_base, SPW),
                                             SPW)])


@jax.jit
def _run(x, batch32, bounds, wb):
    mesh = plsc.VectorSubcoreMesh(core_axis_name="c", subcore_axis_name="s")
    fn = pl.kernel(
        _sc_body,
        out_type=jax.ShapeDtypeStruct((S, 2 * D), jnp.float32),
        mesh=mesh,
        scratch_types=[
            pltpu.VMEM((C, D), jnp.float32),
            pltpu.VMEM((C,), jnp.int32),
            pltpu.VMEM((C, D), jnp.float32),
            pltpu.VMEM((C,), jnp.int32),
            pltpu.VMEM((4 * L,), jnp.float32),
            pltpu.VMEM((D + L,), jnp.float32),
            pltpu.VMEM((SPW + 1, D), jnp.float32),
            pltpu.VMEM((SPW + 1, D), jnp.float32),
            pltpu.VMEM((SPW + 1, L), jnp.float32),
            pltpu.VMEM((SPW, 2 * D), jnp.float32),
            pltpu.SemaphoreType.DMA,
            pltpu.SemaphoreType.DMA,
            pltpu.SemaphoreType.DMA,
            pltpu.SemaphoreType.DMA,
        ],
    )
    return fn(x, batch32, bounds, wb)


def kernel(x, batch, W, b):
    batch32 = batch.astype(jnp.int32)
    targets = jnp.arange(0, S + 1, SPW, dtype=jnp.int32)
    bounds = jnp.searchsorted(batch32, targets).astype(jnp.float32)
    bounds = jnp.concatenate(
        [bounds, jnp.zeros((4 * L - (NW + 1),), jnp.float32)])
    wb = jnp.concatenate([W.reshape(D), jnp.full((L,), b[0], jnp.float32)])
    return _run(x, batch32, bounds, wb)
